# baseline (speedup 1.0000x reference)
import jax
import jax.numpy as jnp
from jax import lax
from jax.experimental import pallas as pl

_IMAGE_W = 1024
_IMAGE_H = 1024
_MIN_SCORE = 0.05
_NMS_THR = 0.5
_MAX_DET = 100
_LANES = 128
_POPS_INLINE = 128
_NEG_INF = float("-inf")


def _decoder_body(cls_ref, reg_ref, anc_ref, s_out_ref, c_out_ref, b_out_ref):
    B, C, R, L = cls_ref.shape
    N = R * L

    def class_step(c, carry):
        m, idx = carry
        v = cls_ref[:, c]
        better = v > m
        return jnp.where(better, v, m), jnp.where(better, c, idx)

    m0 = cls_ref[:, 0]
    idx0 = jnp.zeros((B, R, L), jnp.int32)
    scores, classes = lax.fori_loop(1, C, class_step, (m0, idx0))

    reg = reg_ref[...]
    anc = anc_ref[...]
    ax1, ay1, ax2, ay2 = anc[:, 0], anc[:, 1], anc[:, 2], anc[:, 3]
    aw = ax2 - ax1
    ah = ay2 - ay1
    acx = ax1 + 0.5 * aw
    acy = ay1 + 0.5 * ah
    tx = reg[:, 0] * 0.1
    ty = reg[:, 1] * 0.1
    tw = reg[:, 2] * 0.2
    th = reg[:, 3] * 0.2
    w = jnp.exp(tw) * aw
    h = jnp.exp(th) * ah
    cx = tx * aw + acx
    cy = ty * ah + acy
    xi1 = jnp.maximum((cx - 0.5 * w).astype(jnp.int32), 0)
    yi1 = jnp.maximum((cy - 0.5 * h).astype(jnp.int32), 0)
    xi2 = jnp.minimum((cx + 0.5 * w).astype(jnp.int32), _IMAGE_W - 1)
    yi2 = jnp.minimum((cy + 0.5 * h).astype(jnp.int32), _IMAGE_H - 1)

    apack = (jnp.clip(xi1, 0, 2047)
             | ((jnp.clip(xi2, -2048, 2047) + 2048) << 11)
             | (classes << 23))
    ppack = (jnp.clip(yi1, 0, 2047)
             | ((jnp.clip(yi2, -2048, 2047) + 2048) << 11))

    work0 = jnp.where(scores > _MIN_SCORE, scores, _NEG_INF)
    li2 = (lax.broadcasted_iota(jnp.int32, (R, L), 0) * L
           + lax.broadcasted_iota(jnp.int32, (R, L), 1))
    lane1 = lax.broadcasted_iota(jnp.int32, (1, _LANES), 1)
    apack_b = [apack[b] for b in range(B)]
    ppack_b = [ppack[b] for b in range(B)]

    def pop1(b, st):
        work, ecnt, so, co, kx1, ky1, kx2, ky2, kar = st
        m2 = jnp.max(work, axis=(0, 1), keepdims=True)
        pick = jnp.min(jnp.where(work == m2, li2, N),
                       axis=(0, 1), keepdims=True)
        oh = li2 == pick
        av = jnp.sum(jnp.where(oh, apack_b[b], 0),
                     axis=(0, 1), keepdims=True)
        pv = jnp.sum(jnp.where(oh, ppack_b[b], 0),
                     axis=(0, 1), keepdims=True)
        active = (ecnt < _MAX_DET) & (m2 > _NEG_INF)

        px1 = (av & 2047).astype(jnp.float32)
        px2 = (((av >> 11) & 4095) - 2048).astype(jnp.float32)
        pcls = (av >> 23).astype(jnp.float32)
        py1 = (pv & 2047).astype(jnp.float32)
        py2 = (((pv >> 11) & 4095) - 2048).astype(jnp.float32)
        pa = (px2 - px1) * (py2 - py1)

        xx1 = jnp.maximum(px1, kx1)
        yy1 = jnp.maximum(py1, ky1)
        xx2 = jnp.minimum(px2, kx2)
        yy2 = jnp.minimum(py2, ky2)
        iw = jnp.maximum(xx2 - xx1, 0.0)
        ih = jnp.maximum(yy2 - yy1, 0.0)
        inter = iw * ih
        union = pa + kar - inter
        iou = jnp.where(union > 0, inter / jnp.where(union > 0, union, 1.0), 0.0)
        supv = (iou >= _NMS_THR) & (co == pcls)
        sup = jnp.max(supv.astype(jnp.int32), axis=1, keepdims=True) > 0
        emit = active & (~sup)

        pos = (lane1 == ecnt) & emit
        so = jnp.where(pos, m2, so)
        co = jnp.where(pos, pcls, co)
        kx1 = jnp.where(pos, px1, kx1)
        ky1 = jnp.where(pos, py1, ky1)
        kx2 = jnp.where(pos, px2, kx2)
        ky2 = jnp.where(pos, py2, ky2)
        kar = jnp.where(pos, pa, kar)
        ecnt = ecnt + jnp.where(emit, 1, 0)
        work = jnp.where(oh & active, _NEG_INF, work)
        return work, ecnt, so, co, kx1, ky1, kx2, ky2, kar

    def pop_all(state):
        return tuple(pop1(b, state[b]) for b in range(B))

    def unfinished(state):
        alive = []
        for b in range(B):
            work, ecnt = state[b][0], state[b][1]
            m2 = jnp.max(work, axis=(0, 1), keepdims=True)
            alive.append(((ecnt < _MAX_DET) & (m2 > _NEG_INF))
                         .astype(jnp.int32))
        return jnp.sum(jnp.stack(alive)) > 0

    neg1 = jnp.full((1, _LANES), -1.0, jnp.float32)
    zero = jnp.zeros((1, _LANES), jnp.float32)
    ecnt0 = jnp.zeros((1, 1), jnp.int32)
    state = tuple((work0[b], ecnt0, neg1, neg1, neg1, neg1, neg1, neg1, zero)
                  for b in range(B))
    state = lax.fori_loop(0, _POPS_INLINE, lambda i, s: pop_all(s), state)
    state = lax.while_loop(unfinished, pop_all, state)
    for b in range(B):
        _, _, so, co, kx1, ky1, kx2, ky2, _ = state[b]
        s_out_ref[b:b + 1, :] = so
        c_out_ref[b:b + 1, :] = co
        b_out_ref[b, 0:1, :] = kx1
        b_out_ref[b, 1:2, :] = ky1
        b_out_ref[b, 2:3, :] = kx2
        b_out_ref[b, 3:4, :] = ky2


def _run_decoder(cls4, reg4, anc4):
    B = cls4.shape[0]
    return pl.pallas_call(
        _decoder_body,
        out_shape=[
            jax.ShapeDtypeStruct((B, _LANES), jnp.float32),
            jax.ShapeDtypeStruct((B, _LANES), jnp.float32),
            jax.ShapeDtypeStruct((B, 4, _LANES), jnp.float32),
        ],
    )(cls4, reg4, anc4)


def kernel(cls_heads, reg_heads, batch_anchors):
    cls = jnp.concatenate([cls_heads[i] for i in range(cls_heads.shape[0])], axis=1)
    reg = jnp.concatenate([reg_heads[i] for i in range(reg_heads.shape[0])], axis=1)
    anc = jnp.concatenate([batch_anchors[i] for i in range(batch_anchors.shape[0])], axis=1)
    B, N, C = cls.shape
    NP = -(-N // _LANES) * _LANES
    R = NP // _LANES
    clsT = jnp.pad(jnp.transpose(cls, (0, 2, 1)),
                   ((0, 0), (0, 0), (0, NP - N)), constant_values=-1.0)
    regT = jnp.pad(jnp.transpose(reg, (0, 2, 1)), ((0, 0), (0, 0), (0, NP - N)))
    ancT = jnp.pad(jnp.transpose(anc, (0, 2, 1)), ((0, 0), (0, 0), (0, NP - N)))
    so, co, bo = _run_decoder(clsT.reshape(B, C, R, _LANES),
                              regT.reshape(B, 4, R, _LANES),
                              ancT.reshape(B, 4, R, _LANES))
    s = so[:, :_MAX_DET]
    c = co[:, :_MAX_DET]
    b = jnp.transpose(bo, (0, 2, 1))[:, :_MAX_DET, :]
    return s, c, b


# speedup vs baseline: 2.9165x; 2.9165x over previous
import jax
import jax.numpy as jnp
from jax import lax
from jax.experimental import pallas as pl

_IMAGE_W = 1024
_IMAGE_H = 1024
_MIN_SCORE = 0.05
_NMS_THR = 0.5
_MAX_DET = 100
_LANES = 128
_NEG_INF = float("-inf")


def _decoder_body(cls_ref, reg_ref, anc_ref, s_out_ref, c_out_ref, b_out_ref):
    B, C, R, L = cls_ref.shape
    N = R * L

    def class_step(c, carry):
        m, idx = carry
        v = cls_ref[:, c]
        better = v > m
        return jnp.where(better, v, m), jnp.where(better, c, idx)

    m0 = cls_ref[:, 0]
    idx0 = jnp.zeros((B, R, L), jnp.int32)
    scores, classes = lax.fori_loop(1, C, class_step, (m0, idx0))
    classes_f = classes.astype(jnp.float32)

    reg = reg_ref[...]
    anc = anc_ref[...]
    ax1, ay1, ax2, ay2 = anc[:, 0], anc[:, 1], anc[:, 2], anc[:, 3]
    aw = ax2 - ax1
    ah = ay2 - ay1
    acx = ax1 + 0.5 * aw
    acy = ay1 + 0.5 * ah
    tx = reg[:, 0] * 0.1
    ty = reg[:, 1] * 0.1
    tw = reg[:, 2] * 0.2
    th = reg[:, 3] * 0.2
    w = jnp.exp(tw) * aw
    h = jnp.exp(th) * ah
    cx = tx * aw + acx
    cy = ty * ah + acy
    xi1 = jnp.maximum((cx - 0.5 * w).astype(jnp.int32), 0)
    yi1 = jnp.maximum((cy - 0.5 * h).astype(jnp.int32), 0)
    xi2 = jnp.minimum((cx + 0.5 * w).astype(jnp.int32), _IMAGE_W - 1)
    yi2 = jnp.minimum((cy + 0.5 * h).astype(jnp.int32), _IMAGE_H - 1)
    bx1 = xi1.astype(jnp.float32)
    by1 = yi1.astype(jnp.float32)
    bx2 = xi2.astype(jnp.float32)
    by2 = yi2.astype(jnp.float32)
    areas = (bx2 - bx1) * (by2 - by1)

    apack = (jnp.clip(xi1, 0, 2047)
             | ((jnp.clip(xi2, -2048, 2047) + 2048) << 11)
             | (classes << 23))
    ppack = (jnp.clip(yi1, 0, 2047)
             | ((jnp.clip(yi2, -2048, 2047) + 2048) << 11))

    work = jnp.where(scores > _MIN_SCORE, scores, _NEG_INF)
    li = (lax.broadcasted_iota(jnp.int32, (B, R, L), 1) * L
          + lax.broadcasted_iota(jnp.int32, (B, R, L), 2))
    lane = lax.broadcasted_iota(jnp.int32, (B, _LANES), 1)

    def step(i, carry):
        work, so, co, o0, o1, o2, o3 = carry
        m = jnp.max(work, axis=(1, 2), keepdims=True)
        pick = jnp.min(jnp.where(work == m, li, N), axis=(1, 2), keepdims=True)
        oh = li == pick

        av = jnp.sum(jnp.where(oh, apack, 0), axis=(1, 2), keepdims=True)
        pv = jnp.sum(jnp.where(oh, ppack, 0), axis=(1, 2), keepdims=True)
        px1 = (av & 2047).astype(jnp.float32)
        px2 = (((av >> 11) & 4095) - 2048).astype(jnp.float32)
        cm = (av >> 23).astype(jnp.float32)
        py1 = (pv & 2047).astype(jnp.float32)
        py2 = (((pv >> 11) & 4095) - 2048).astype(jnp.float32)
        pa = (px2 - px1) * (py2 - py1)
        xx1 = jnp.maximum(px1, bx1)
        yy1 = jnp.maximum(py1, by1)
        xx2 = jnp.minimum(px2, bx2)
        yy2 = jnp.minimum(py2, by2)
        iw = jnp.maximum(xx2 - xx1, 0.0)
        ih = jnp.maximum(yy2 - yy1, 0.0)
        inter = iw * ih
        union = pa + areas - inter
        iou = jnp.where(union > 0, inter / jnp.where(union > 0, union, 1.0), 0.0)
        sup = (iou >= _NMS_THR) & (classes_f == cm)
        valid = m > _NEG_INF
        work = jnp.where(valid & (sup | oh), _NEG_INF, work)

        pos = (lane == i) & valid[:, :, 0]
        so = jnp.where(pos, m[:, :, 0], so)
        co = jnp.where(pos, cm[:, :, 0], co)
        o0 = jnp.where(pos, px1[:, :, 0], o0)
        o1 = jnp.where(pos, py1[:, :, 0], o1)
        o2 = jnp.where(pos, px2[:, :, 0], o2)
        o3 = jnp.where(pos, py2[:, :, 0], o3)
        return work, so, co, o0, o1, o2, o3

    neg1 = jnp.full((B, _LANES), -1.0, jnp.float32)
    carry = lax.fori_loop(0, _MAX_DET, step,
                          (work, neg1, neg1, neg1, neg1, neg1, neg1))
    _, so, co, o0, o1, o2, o3 = carry
    s_out_ref[...] = so
    c_out_ref[...] = co
    b_out_ref[:, 0, :] = o0
    b_out_ref[:, 1, :] = o1
    b_out_ref[:, 2, :] = o2
    b_out_ref[:, 3, :] = o3


def _run_decoder(cls4, reg4, anc4):
    B = cls4.shape[0]
    return pl.pallas_call(
        _decoder_body,
        out_shape=[
            jax.ShapeDtypeStruct((B, _LANES), jnp.float32),
            jax.ShapeDtypeStruct((B, _LANES), jnp.float32),
            jax.ShapeDtypeStruct((B, 4, _LANES), jnp.float32),
        ],
    )(cls4, reg4, anc4)


def kernel(cls_heads, reg_heads, batch_anchors):
    cls = jnp.concatenate([cls_heads[i] for i in range(cls_heads.shape[0])], axis=1)
    reg = jnp.concatenate([reg_heads[i] for i in range(reg_heads.shape[0])], axis=1)
    anc = jnp.concatenate([batch_anchors[i] for i in range(batch_anchors.shape[0])], axis=1)
    B, N, C = cls.shape
    NP = -(-N // _LANES) * _LANES
    R = NP // _LANES
    clsT = jnp.pad(jnp.transpose(cls, (0, 2, 1)),
                   ((0, 0), (0, 0), (0, NP - N)), constant_values=-1.0)
    regT = jnp.pad(jnp.transpose(reg, (0, 2, 1)), ((0, 0), (0, 0), (0, NP - N)))
    ancT = jnp.pad(jnp.transpose(anc, (0, 2, 1)), ((0, 0), (0, 0), (0, NP - N)))
    so, co, bo = _run_decoder(clsT.reshape(B, C, R, _LANES),
                              regT.reshape(B, 4, R, _LANES),
                              ancT.reshape(B, 4, R, _LANES))
    s = so[:, :_MAX_DET]
    c = co[:, :_MAX_DET]
    b = jnp.transpose(bo, (0, 2, 1))[:, :_MAX_DET, :]
    return s, c, b


# speedup vs baseline: 3.2286x; 1.1070x over previous
import jax
import jax.numpy as jnp
from jax import lax
from jax.experimental import pallas as pl

_IMAGE_W = 1024
_IMAGE_H = 1024
_MIN_SCORE = 0.05
_NMS_THR = 0.5
_MAX_DET = 100
_LANES = 128
_NEG_INF = float("-inf")


def _decoder_body(cls_ref, reg_ref, anc_ref, s_out_ref, c_out_ref, b_out_ref):
    B, C, R, L = cls_ref.shape
    N = R * L

    def class_step(c, carry):
        m, idx = carry
        v = cls_ref[:, c]
        better = v > m
        return jnp.where(better, v, m), jnp.where(better, c, idx)

    m0 = cls_ref[:, 0]
    idx0 = jnp.zeros((B, R, L), jnp.int32)
    scores, classes = lax.fori_loop(1, C, class_step, (m0, idx0))
    classes_f = classes.astype(jnp.float32)

    reg = reg_ref[...]
    anc = anc_ref[...]
    ax1, ay1, ax2, ay2 = anc[:, 0], anc[:, 1], anc[:, 2], anc[:, 3]
    aw = ax2 - ax1
    ah = ay2 - ay1
    acx = ax1 + 0.5 * aw
    acy = ay1 + 0.5 * ah
    tx = reg[:, 0] * 0.1
    ty = reg[:, 1] * 0.1
    tw = reg[:, 2] * 0.2
    th = reg[:, 3] * 0.2
    w = jnp.exp(tw) * aw
    h = jnp.exp(th) * ah
    cx = tx * aw + acx
    cy = ty * ah + acy
    xi1 = jnp.maximum((cx - 0.5 * w).astype(jnp.int32), 0)
    yi1 = jnp.maximum((cy - 0.5 * h).astype(jnp.int32), 0)
    xi2 = jnp.minimum((cx + 0.5 * w).astype(jnp.int32), _IMAGE_W - 1)
    yi2 = jnp.minimum((cy + 0.5 * h).astype(jnp.int32), _IMAGE_H - 1)
    bx1 = xi1.astype(jnp.float32)
    by1 = yi1.astype(jnp.float32)
    bx2 = xi2.astype(jnp.float32)
    by2 = yi2.astype(jnp.float32)
    areas = (bx2 - bx1) * (by2 - by1)

    apack = (jnp.clip(xi1, 0, 2047)
             | ((jnp.clip(xi2, -2048, 2047) + 2048) << 11)
             | (classes << 23))
    ppack = (jnp.clip(yi1, 0, 2047)
             | ((jnp.clip(yi2, -2048, 2047) + 2048) << 11))

    work = jnp.where(scores > _MIN_SCORE, scores, _NEG_INF)
    li = (lax.broadcasted_iota(jnp.int32, (B, R, L), 1) * L
          + lax.broadcasted_iota(jnp.int32, (B, R, L), 2))
    lane = lax.broadcasted_iota(jnp.int32, (B, _LANES), 1)

    K = 4

    def step(carry):
        work, ecnt, so, co, o0, o1, o2, o3 = carry
        cur = work
        ms, ohs, valids, fields = [], [], [], []
        for _ in range(K):
            m = jnp.max(cur, axis=(1, 2), keepdims=True)
            pick = jnp.min(jnp.where(cur == m, li, N),
                           axis=(1, 2), keepdims=True)
            oh = li == pick
            cur = jnp.where(oh, _NEG_INF, cur)
            av = jnp.sum(jnp.where(oh, apack, 0), axis=(1, 2), keepdims=True)
            pv = jnp.sum(jnp.where(oh, ppack, 0), axis=(1, 2), keepdims=True)
            px1 = (av & 2047).astype(jnp.float32)
            px2 = (((av >> 11) & 4095) - 2048).astype(jnp.float32)
            cm = (av >> 23).astype(jnp.float32)
            py1 = (pv & 2047).astype(jnp.float32)
            py2 = (((pv >> 11) & 4095) - 2048).astype(jnp.float32)
            pa = (px2 - px1) * (py2 - py1)
            ms.append(m)
            ohs.append(oh)
            valids.append(m > _NEG_INF)
            fields.append((px1, py1, px2, py2, pa, cm))

        def pair_iou(fi, fj):
            xx1 = jnp.maximum(fi[0], fj[0])
            yy1 = jnp.maximum(fi[1], fj[1])
            xx2 = jnp.minimum(fi[2], fj[2])
            yy2 = jnp.minimum(fi[3], fj[3])
            inter = jnp.maximum(xx2 - xx1, 0.0) * jnp.maximum(yy2 - yy1, 0.0)
            union = fi[4] + fj[4] - inter
            return jnp.where(union > 0,
                             inter / jnp.where(union > 0, union, 1.0), 0.0)

        emits = []
        kill = ohs[0] & valids[0]
        cnt = ecnt
        for j in range(K):
            supped = jnp.zeros_like(valids[0][:, :, 0])
            for i2 in range(j):
                iou_ij = pair_iou(fields[i2], fields[j])
                supped = supped | (emits[i2]
                                   & (fields[i2][5] == fields[j][5])[:, :, 0]
                                   & (iou_ij >= _NMS_THR)[:, :, 0])
            emit = valids[j][:, :, 0] & (cnt < _MAX_DET) & (~supped)
            emits.append(emit)
            px1, py1, px2, py2, pa, cm = fields[j]
            pos = (lane == cnt) & emit
            so = jnp.where(pos, ms[j][:, :, 0], so)
            co = jnp.where(pos, cm[:, :, 0], co)
            o0 = jnp.where(pos, px1[:, :, 0], o0)
            o1 = jnp.where(pos, py1[:, :, 0], o1)
            o2 = jnp.where(pos, px2[:, :, 0], o2)
            o3 = jnp.where(pos, py2[:, :, 0], o3)
            cnt = cnt + jnp.where(emit, 1, 0)
            if j > 0:
                kill = kill | (ohs[j] & valids[j])

        for j in range(K):
            px1, py1, px2, py2, pa, cm = fields[j]
            xx1 = jnp.maximum(px1, bx1)
            yy1 = jnp.maximum(py1, by1)
            xx2 = jnp.minimum(px2, bx2)
            yy2 = jnp.minimum(py2, by2)
            iw = jnp.maximum(xx2 - xx1, 0.0)
            ih = jnp.maximum(yy2 - yy1, 0.0)
            inter = iw * ih
            union = pa + areas - inter
            iou = jnp.where(union > 0,
                            inter / jnp.where(union > 0, union, 1.0), 0.0)
            sup = (iou >= _NMS_THR) & (classes_f == cm)
            kill = kill | (sup & emits[j][:, :, None])
        work = jnp.where(kill, _NEG_INF, work)
        return work, cnt, so, co, o0, o1, o2, o3

    def unfinished(carry):
        work, ecnt = carry[0], carry[1]
        m = jnp.max(work, axis=(1, 2), keepdims=True)[:, :, 0]
        active = (ecnt < _MAX_DET) & (m > _NEG_INF)
        return jnp.sum(active.astype(jnp.int32)) > 0

    neg1 = jnp.full((B, _LANES), -1.0, jnp.float32)
    ecnt0 = jnp.zeros((B, 1), jnp.int32)
    carry = (work, ecnt0, neg1, neg1, neg1, neg1, neg1, neg1)
    carry = lax.fori_loop(0, _MAX_DET // K, lambda i, c: step(c), carry)
    carry = lax.while_loop(unfinished, step, carry)
    _, _, so, co, o0, o1, o2, o3 = carry
    s_out_ref[...] = so
    c_out_ref[...] = co
    b_out_ref[:, 0, :] = o0
    b_out_ref[:, 1, :] = o1
    b_out_ref[:, 2, :] = o2
    b_out_ref[:, 3, :] = o3


def _run_decoder(cls4, reg4, anc4):
    B = cls4.shape[0]
    return pl.pallas_call(
        _decoder_body,
        out_shape=[
            jax.ShapeDtypeStruct((B, _LANES), jnp.float32),
            jax.ShapeDtypeStruct((B, _LANES), jnp.float32),
            jax.ShapeDtypeStruct((B, 4, _LANES), jnp.float32),
        ],
    )(cls4, reg4, anc4)


def kernel(cls_heads, reg_heads, batch_anchors):
    cls = jnp.concatenate([cls_heads[i] for i in range(cls_heads.shape[0])], axis=1)
    reg = jnp.concatenate([reg_heads[i] for i in range(reg_heads.shape[0])], axis=1)
    anc = jnp.concatenate([batch_anchors[i] for i in range(batch_anchors.shape[0])], axis=1)
    B, N, C = cls.shape
    NP = -(-N // _LANES) * _LANES
    R = NP // _LANES
    clsT = jnp.pad(jnp.transpose(cls, (0, 2, 1)),
                   ((0, 0), (0, 0), (0, NP - N)), constant_values=-1.0)
    regT = jnp.pad(jnp.transpose(reg, (0, 2, 1)), ((0, 0), (0, 0), (0, NP - N)))
    ancT = jnp.pad(jnp.transpose(anc, (0, 2, 1)), ((0, 0), (0, 0), (0, NP - N)))
    so, co, bo = _run_decoder(clsT.reshape(B, C, R, _LANES),
                              regT.reshape(B, 4, R, _LANES),
                              ancT.reshape(B, 4, R, _LANES))
    s = so[:, :_MAX_DET]
    c = co[:, :_MAX_DET]
    b = jnp.transpose(bo, (0, 2, 1))[:, :_MAX_DET, :]
    return s, c, b
